# baseline (speedup 1.0000x reference)
import functools

import jax
import jax.numpy as jnp
from jax import lax
from jax.experimental import pallas as pl
from jax.experimental.pallas import tpu as pltpu
from jax.experimental.pallas import tpu_sc as plsc

VOCAB = 1000
VPAD = 1024
VMAIN = 896
VTAIL = VOCAB - VMAIN
HIDDEN = 128
BATCH = 1024
SEQ = 50
SPAD = 56
SMAIN = 48
HIDDEN_ = HIDDEN
NW = 32
BPW = BATCH // NW


def _logit_table_body(emb_ref, w_ref, b_ref, m_ref):
    m_ref[...] = lax.dot_general(
        emb_ref[...], w_ref[...],
        dimension_numbers=(((1,), (1,)), ((), ())),
        preferred_element_type=jnp.float32,
    ) + b_ref[...]


def _logit_table(emb_table, proj_w_pad, proj_b_pad):
    return pl.pallas_call(
        _logit_table_body,
        out_shape=jax.ShapeDtypeStruct((VOCAB, VPAD), jnp.float32),
    )(emb_table, proj_w_pad, proj_b_pad)


@functools.cache
def _gather_logits():
    mesh = plsc.VectorSubcoreMesh(core_axis_name="c", subcore_axis_name="s")

    @functools.partial(
        pl.kernel,
        mesh=mesh,
        out_type=jax.ShapeDtypeStruct((BATCH, SEQ, VOCAB), jnp.float32),
        scratch_types=[
            pltpu.VMEM((BPW * SPAD,), jnp.int32),
            pltpu.VMEM((SPAD, VPAD), jnp.float32),
            pltpu.VMEM((SMAIN, VTAIL), jnp.float32),
            pltpu.VMEM((SEQ - SMAIN, VOCAB), jnp.float32),
            pltpu.SemaphoreType.DMA,
            pltpu.SemaphoreType.DMA,
        ],
    )
    def gather(m_hbm, idx_hbm, out_hbm, idx_v, rows_v, tail_v, last_v, sem, wsem):
        wid = lax.axis_index("s") * 2 + lax.axis_index("c")
        base = wid * BPW
        pltpu.sync_copy(idx_hbm.at[pl.ds(base * SPAD, BPW * SPAD)], idx_v)

        def narrow_row(r, carry):
            for j in range(6):
                tail_v[r, pl.ds(16 * j, 16)] = rows_v[r, pl.ds(VMAIN + 16 * j, 16)]
            tail_v[r, pl.ds(VTAIL - 16, 16)] = rows_v[r, pl.ds(VMAIN + VTAIL - 16, 16)]
            return carry

        def body(g, carry):
            b = base + g
            pltpu.async_copy(
                m_hbm.at[idx_v.at[pl.ds(g * SPAD, SPAD)]], rows_v, sem
            ).wait()
            cp = pltpu.async_copy(
                rows_v.at[pl.ds(0, SMAIN), pl.ds(0, VMAIN)],
                out_hbm.at[b, pl.ds(0, SMAIN), pl.ds(0, VMAIN)],
                wsem,
            )
            lax.fori_loop(0, SMAIN, narrow_row, 0)
            for r in range(SMAIN, SEQ):
                for j in range(VOCAB // 16):
                    last_v[r - SMAIN, pl.ds(16 * j, 16)] = rows_v[r, pl.ds(16 * j, 16)]
                last_v[r - SMAIN, pl.ds(VOCAB - 16, 16)] = rows_v[r, pl.ds(VOCAB - 16, 16)]
            pltpu.sync_copy(tail_v, out_hbm.at[b, pl.ds(0, SMAIN), pl.ds(VMAIN, VTAIL)])
            pltpu.sync_copy(last_v, out_hbm.at[b, pl.ds(SMAIN, SEQ - SMAIN)])
            cp.wait()
            return carry

        lax.fori_loop(0, BPW, body, 0)

    return gather


def kernel(input_ids, emb_table, proj_w, proj_b):
    w_pad = jnp.pad(proj_w, ((0, VPAD - VOCAB), (0, 0)))
    b_pad = jnp.pad(proj_b, (0, VPAD - VOCAB)).reshape(1, VPAD)
    m = _logit_table(emb_table, w_pad, b_pad)
    ids = jnp.pad(input_ids.astype(jnp.int32), ((0, 0), (0, SPAD - SEQ)))
    out = _gather_logits()(m, ids.reshape(-1))
    return out


# speedup vs baseline: 1.0136x; 1.0136x over previous
import functools

import jax
import jax.numpy as jnp
from jax import lax
from jax.experimental import pallas as pl
from jax.experimental.pallas import tpu as pltpu
from jax.experimental.pallas import tpu_sc as plsc

VOCAB = 1000
VPAD = 1024
VMAIN = 896
VTAIL = VOCAB - VMAIN
HIDDEN = 128
BATCH = 1024
SEQ = 50
SPAD = 56
SMAIN = 48
NW = 32
BPW = BATCH // NW


def _logit_table_body(emb_ref, w_ref, b_ref, m_ref):
    m_ref[...] = lax.dot_general(
        emb_ref[...], w_ref[...],
        dimension_numbers=(((1,), (1,)), ((), ())),
        preferred_element_type=jnp.float32,
    ) + b_ref[...]


def _logit_table(emb_table, proj_w_pad, proj_b_pad):
    return pl.pallas_call(
        _logit_table_body,
        out_shape=jax.ShapeDtypeStruct((VOCAB, VPAD), jnp.float32),
    )(emb_table, proj_w_pad, proj_b_pad)


@functools.cache
def _gather_logits():
    mesh = plsc.VectorSubcoreMesh(core_axis_name="c", subcore_axis_name="s")

    @functools.partial(
        pl.kernel,
        mesh=mesh,
        out_type=jax.ShapeDtypeStruct((BATCH, SEQ, VOCAB), jnp.float32),
        scratch_types=[
            pltpu.VMEM((BPW * SPAD,), jnp.int32),
            pltpu.VMEM((SPAD, VPAD), jnp.float32),
            pltpu.VMEM((SPAD, VPAD), jnp.float32),
            pltpu.VMEM((SMAIN, VTAIL), jnp.float32),
            pltpu.VMEM((SEQ - SMAIN, VOCAB), jnp.float32),
            pltpu.SemaphoreType.DMA,
            pltpu.SemaphoreType.DMA,
            pltpu.SemaphoreType.DMA,
        ],
    )
    def gather(m_hbm, idx_hbm, out_hbm, idx_v, rows0, rows1, tail_v, last_v,
               gsem, wsem, esem):
        wid = lax.axis_index("s") * 2 + lax.axis_index("c")
        base = wid * BPW
        pltpu.sync_copy(idx_hbm.at[pl.ds(base * SPAD, BPW * SPAD)], idx_v)

        def gather_desc(g, buf):
            return pltpu.make_async_copy(
                m_hbm.at[idx_v.at[pl.ds(g * SPAD, SPAD)]], buf, gsem
            )

        def write1_desc(g, buf):
            return pltpu.make_async_copy(
                buf.at[pl.ds(0, SMAIN), pl.ds(0, VMAIN)],
                out_hbm.at[base + g, pl.ds(0, SMAIN), pl.ds(0, VMAIN)],
                wsem,
            )

        def tail_desc(g):
            return pltpu.make_async_copy(
                tail_v,
                out_hbm.at[base + g, pl.ds(0, SMAIN), pl.ds(VMAIN, VTAIL)],
                esem,
            )

        def last_desc(g):
            return pltpu.make_async_copy(
                last_v,
                out_hbm.at[base + g, pl.ds(SMAIN, SEQ - SMAIN)],
                esem,
            )

        def narrow(buf):
            def narrow_row(r, carry):
                for j in range(6):
                    tail_v[r, pl.ds(16 * j, 16)] = buf[r, pl.ds(VMAIN + 16 * j, 16)]
                tail_v[r, pl.ds(VTAIL - 16, 16)] = buf[r, pl.ds(VMAIN + VTAIL - 16, 16)]
                return carry

            lax.fori_loop(0, SMAIN, narrow_row, 0)
            for r in range(SMAIN, SEQ):
                for j in range(VOCAB // 16):
                    last_v[r - SMAIN, pl.ds(16 * j, 16)] = buf[r, pl.ds(16 * j, 16)]
                last_v[r - SMAIN, pl.ds(VOCAB - 16, 16)] = buf[r, pl.ds(VOCAB - 16, 16)]

        gather_desc(0, rows0).start()

        def body(k, carry):
            for half in range(2):
                g = 2 * k + half
                buf = rows0 if half == 0 else rows1
                nbuf = rows1 if half == 0 else rows0
                gather_desc(g, buf).wait()

                @pl.when(g >= 1)
                def _():
                    write1_desc(g - 1, nbuf).wait()

                @pl.when(g < BPW - 1)
                def _():
                    gather_desc(g + 1, nbuf).start()

                write1_desc(g, buf).start()

                @pl.when(g >= 1)
                def _():
                    tail_desc(g - 1).wait()
                    last_desc(g - 1).wait()

                narrow(buf)
                tail_desc(g).start()
                last_desc(g).start()
            return carry

        lax.fori_loop(0, BPW // 2, body, 0)
        write1_desc(BPW - 1, rows1).wait()
        tail_desc(BPW - 1).wait()
        last_desc(BPW - 1).wait()

    return gather


def kernel(input_ids, emb_table, proj_w, proj_b):
    w_pad = jnp.pad(proj_w, ((0, VPAD - VOCAB), (0, 0)))
    b_pad = jnp.pad(proj_b, (0, VPAD - VOCAB)).reshape(1, VPAD)
    m = _logit_table(emb_table, w_pad, b_pad)
    ids = jnp.pad(input_ids.astype(jnp.int32), ((0, 0), (0, SPAD - SEQ)))
    out = _gather_logits()(m, ids.reshape(-1))
    return out


# speedup vs baseline: 2.0433x; 2.0159x over previous
import functools

import jax
import jax.numpy as jnp
from jax import lax
from jax.experimental import pallas as pl
from jax.experimental.pallas import tpu as pltpu
from jax.experimental.pallas import tpu_sc as plsc

VOCAB = 1000
VPAD = 1024
VMAIN = 896
VTAIL = VOCAB - VMAIN
HIDDEN = 128
BATCH = 1024
SEQ = 50
SPAD = 56
SMAIN = 48
NW = 32
BPW = BATCH // NW


def _logit_table_body(emb_ref, w_ref, b_ref, m_ref):
    m_ref[...] = lax.dot_general(
        emb_ref[...], w_ref[...],
        dimension_numbers=(((1,), (1,)), ((), ())),
        preferred_element_type=jnp.float32,
    ) + b_ref[...]


def _logit_table(emb_table, proj_w_pad, proj_b_pad):
    return pl.pallas_call(
        _logit_table_body,
        out_shape=jax.ShapeDtypeStruct((VOCAB, VPAD), jnp.float32),
    )(emb_table, proj_w_pad, proj_b_pad)


@functools.cache
def _gather_logits():
    mesh = plsc.VectorSubcoreMesh(core_axis_name="c", subcore_axis_name="s")

    @functools.partial(
        pl.kernel,
        mesh=mesh,
        out_type=jax.ShapeDtypeStruct((BATCH, SEQ, VOCAB), jnp.float32),
        scratch_types=[
            pltpu.VMEM((BPW * SPAD,), jnp.int32),
            pltpu.VMEM((SPAD, VPAD), jnp.float32),
            pltpu.VMEM((SPAD, VPAD), jnp.float32),
            pltpu.VMEM((SMAIN, VTAIL), jnp.float32),
            pltpu.VMEM((SEQ - SMAIN, VOCAB), jnp.float32),
            pltpu.SemaphoreType.DMA,
            pltpu.SemaphoreType.DMA,
            pltpu.SemaphoreType.DMA,
        ],
    )
    def gather(m_hbm, idx_hbm, out_hbm, idx_v, rows0, rows1, tail_v, last_v,
               gsem, wsem, esem):
        wid = lax.axis_index("s") * 2 + lax.axis_index("c")
        base = wid * BPW
        pltpu.sync_copy(idx_hbm.at[pl.ds(base * SPAD, BPW * SPAD)], idx_v)

        def gather_desc(g, buf):
            return pltpu.make_async_copy(
                m_hbm.at[idx_v.at[pl.ds(g * SPAD, SPAD)]], buf, gsem
            )

        def write1_desc(g, buf):
            return pltpu.make_async_copy(
                buf.at[pl.ds(0, SMAIN), pl.ds(0, VMAIN)],
                out_hbm.at[base + g, pl.ds(0, SMAIN), pl.ds(0, VMAIN)],
                wsem,
            )

        def tail_desc(g):
            return pltpu.make_async_copy(
                tail_v,
                out_hbm.at[base + g, pl.ds(0, SMAIN), pl.ds(VMAIN, VTAIL)],
                esem,
            )

        def last_desc(g):
            return pltpu.make_async_copy(
                last_v,
                out_hbm.at[base + g, pl.ds(SMAIN, SEQ - SMAIN)],
                esem,
            )

        def narrow(buf):
            def narrow_row(r, carry):
                for j in range(6):
                    tail_v[r, pl.ds(16 * j, 16)] = buf[r, pl.ds(VMAIN + 16 * j, 16)]
                tail_v[r, pl.ds(VTAIL - 16, 16)] = buf[r, pl.ds(VMAIN + VTAIL - 16, 16)]
                return carry

            lax.fori_loop(0, SMAIN, narrow_row, 0)
            for r in range(SMAIN, SEQ):
                for j in range(VOCAB // 16):
                    last_v[r - SMAIN, pl.ds(16 * j, 16)] = buf[r, pl.ds(16 * j, 16)]
                last_v[r - SMAIN, pl.ds(VOCAB - 16, 16)] = buf[r, pl.ds(VOCAB - 16, 16)]

        gather_desc(0, rows0).start()

        def body(k, carry):
            for half in range(2):
                g = 2 * k + half
                buf = rows0 if half == 0 else rows1
                nbuf = rows1 if half == 0 else rows0
                gather_desc(g, buf).wait()

                @pl.when(g >= 1)
                def _():
                    write1_desc(g - 1, nbuf).wait()

                @pl.when(g < BPW - 1)
                def _():
                    gather_desc(g + 1, nbuf).start()

                write1_desc(g, buf).start()

                @pl.when(g >= 1)
                def _():
                    tail_desc(g - 1).wait()
                    last_desc(g - 1).wait()

                narrow(buf)
                tail_desc(g).start()
                last_desc(g).start()
            return carry

        lax.fori_loop(0, BPW // 2, body, 0)
        write1_desc(BPW - 1, rows1).wait()
        tail_desc(BPW - 1).wait()
        last_desc(BPW - 1).wait()

    return gather


def kernel(input_ids, emb_table, proj_w, proj_b):
    w_pad = jnp.pad(proj_w, ((0, VPAD - VOCAB), (0, 0)))
    b_pad = jnp.pad(proj_b, (0, VPAD - VOCAB)).reshape(1, VPAD)
    m = _logit_table(emb_table, w_pad, b_pad)
    pads = (jnp.arange(BATCH, dtype=jnp.int32)[:, None] * (SPAD - SEQ)
            + jnp.arange(SPAD - SEQ, dtype=jnp.int32)[None, :]) % VOCAB
    ids = jnp.concatenate([input_ids.astype(jnp.int32), pads], axis=1)
    out = _gather_logits()(m, ids.reshape(-1))
    return out
